# baseline (speedup 1.0000x reference)
import functools

import jax
import jax.numpy as jnp
from jax import lax
from jax.experimental import pallas as pl
from jax.experimental.pallas import tpu as pltpu
from jax.experimental.pallas import tpu_sc as plsc

NB = 16
NC, NS = 2, 16
NW = NC * NS
CHUNK = 128


def _silu(x):
    return x * jax.nn.sigmoid(x)


def _dot(a, b):
    return jnp.dot(a, b, preferred_element_type=jnp.float32)


def _dotT(a, b):
    return lax.dot_general(a, b, (((0,), (0,)), ((), ())),
                           preferred_element_type=jnp.float32)


def _full(shape):
    return pl.BlockSpec(shape, lambda i: (0,) * len(shape))


def _rows(r, c):
    return pl.BlockSpec((r, c), lambda i: (i, 0))


def _pre_body(nodes_ref, xp_ref, nb_ref, wn_ref, wx_ref, b1_ref, w2_ref,
              b2_ref, ws_ref, wd_ref, be_ref, h_ref, ts_ref, td_ref, oh_ref):
    r = nodes_ref.shape[0]
    hmid = _silu(_dot(nodes_ref[...], wn_ref[...])
                 + _dot(xp_ref[...], wx_ref[...]) + b1_ref[...])
    h = _dot(hmid, w2_ref[...]) + b2_ref[...]
    h_ref[...] = h
    cols = lax.broadcasted_iota(jnp.int32, (r, NB), 1)
    oh = (nb_ref[...] == cols).astype(jnp.float32)
    oh_ref[...] = oh
    z = jnp.zeros((r, 128 - 64 - NB), jnp.float32)
    ts_ref[...] = jnp.concatenate(
        [_dot(h, ws_ref[...]) + be_ref[...], oh, z], axis=1)
    td_ref[...] = jnp.concatenate(
        [_dot(h, wd_ref[...]), jnp.zeros((r, 64), jnp.float32)], axis=1)


def _run_pre(nodes, xp, nb2, wn, wx, b1, w2, b2, ws, wd, be):
    n = nodes.shape[0]
    r = 2000
    return pl.pallas_call(
        _pre_body,
        grid=(n // r,),
        in_specs=[_rows(r, 128), _rows(r, 128), _rows(r, 1),
                  _full(wn.shape), _full(wx.shape), _full(b1.shape),
                  _full(w2.shape), _full(b2.shape), _full(ws.shape),
                  _full(wd.shape), _full(be.shape)],
        out_specs=[_rows(r, 128), _rows(r, 128), _rows(r, 128), _rows(r, NB)],
        out_shape=[jax.ShapeDtypeStruct((n, 128), jnp.float32),
                   jax.ShapeDtypeStruct((n, 128), jnp.float32),
                   jax.ShapeDtypeStruct((n, 128), jnp.float32),
                   jax.ShapeDtypeStruct((n, NB), jnp.float32)],
    )(nodes, xp, nb2, wn, wx, b1, w2, b2, ws, wd, be)


def _edge_body(megnet, gs_ref, gd_ref, edges_ref, we_ref, w2_ref, b2_ref,
               *refs):
    if megnet:
        wv_ref, virt_ref = refs[0], refs[1]
        enew_ref, sum_eb_ref = refs[2], refs[3]
    else:
        enew_ref, sum_eb_ref, cnt_eb_ref = refs[:3]
    gs = gs_ref[...]
    oh = gs[:, 64:64 + NB]
    pre = gs[:, 0:64] + gd_ref[...][:, 0:64] + _dot(edges_ref[...], we_ref[...])
    if megnet:
        pre = pre + _dot(oh, _dot(virt_ref[...], wv_ref[...]))
    eo = _dot(_silu(pre), w2_ref[...]) + b2_ref[...]
    enew_ref[...] = edges_ref[...] + eo

    @pl.when(pl.program_id(0) == 0)
    def _():
        sum_eb_ref[...] = jnp.zeros_like(sum_eb_ref)
        if not megnet:
            cnt_eb_ref[...] = jnp.zeros_like(cnt_eb_ref)

    sum_eb_ref[...] += _dotT(oh, eo)
    if not megnet:
        cnt_eb_ref[...] += _dotT(oh, jnp.ones_like(eo))


def _run_edge(megnet, gs, gd, edges, we, w2, b2, wv=None, virt=None, part=0,
              nparts=1):
    e = edges.shape[0]
    r = 8000
    grid = e // nparts // r
    off = part * grid
    rows = pl.BlockSpec((r, 128), lambda i: (i + off, 0))
    in_specs = [rows, rows, rows,
                _full(we.shape), _full(w2.shape), _full(b2.shape)]
    args = [gs, gd, edges, we, w2, b2]
    if megnet:
        in_specs += [_full(wv.shape), _full(virt.shape)]
        args += [wv, virt]
    out_specs = [rows, _full((NB, 128))]
    out_shape = [jax.ShapeDtypeStruct((e, 128), jnp.float32),
                 jax.ShapeDtypeStruct((NB, 128), jnp.float32)]
    if not megnet:
        out_specs.append(_full((NB, 128)))
        out_shape.append(jax.ShapeDtypeStruct((NB, 128), jnp.float32))
    return pl.pallas_call(
        functools.partial(_edge_body, megnet),
        grid=(grid,),
        in_specs=in_specs,
        out_specs=out_specs,
        out_shape=out_shape,
    )(*args)


def _node_body(first, na, nb, h_ref, *refs):
    pa = refs[:na]
    pb = refs[na:na + nb]
    (c0_ref, c1_ref, oh_ref, wh_ref, wa_ref, b1_ref, w2_ref,
     b2_ref) = refs[na + nb:na + nb + 8]
    refs = refs[na + nb + 8:]
    if first:
        wsn_ref, wdn_ref, ben_ref = refs[0], refs[1], refs[2]
        hnew_ref, sumn_ref, cntn_ref, ts_ref, td_ref = refs[3:]
    else:
        wv_ref, virt_ref = refs[0], refs[1]
        hnew_ref, sumn_ref = refs[2], refs[3]
    oh = oh_ref[...]
    cnt = jnp.maximum(c0_ref[...] + c1_ref[...], 1.0)
    sa = sum(p[...] for p in pa[1:]) + pa[0][...]
    sb = sum(p[...] for p in pb[1:]) + pb[0][...]
    agg = (sa - sb) / cnt
    pre = _dot(h_ref[...], wh_ref[...]) + _dot(agg, wa_ref[...]) + b1_ref[...]
    if not first:
        pre = pre + _dot(oh, _dot(virt_ref[...], wv_ref[...]))
    nout = _dot(_silu(pre), w2_ref[...]) + b2_ref[...]
    hnew = h_ref[...] + nout
    hnew_ref[...] = hnew

    @pl.when(pl.program_id(0) == 0)
    def _():
        sumn_ref[...] = jnp.zeros_like(sumn_ref)
        if first:
            cntn_ref[...] = jnp.zeros_like(cntn_ref)

    sumn_ref[...] += _dotT(oh, nout)
    if first:
        cntn_ref[...] += _dotT(oh, jnp.ones_like(nout))
        r = oh.shape[0]
        z = jnp.zeros((r, 128 - 64 - NB), jnp.float32)
        ts_ref[...] = jnp.concatenate(
            [_dot(hnew, wsn_ref[...]) + ben_ref[...], oh, z], axis=1)
        td_ref[...] = jnp.concatenate(
            [_dot(hnew, wdn_ref[...]), jnp.zeros((r, 64), jnp.float32)],
            axis=1)


def _run_node(first, h, pa, pb, c0, c1, ohn, wh, wa, b1, w2, b2, extra):
    n = h.shape[0]
    r = 2000
    npart = len(pa) + len(pb)
    in_specs = [_rows(r, 128)] * (3 + npart) + [_rows(r, NB),
                                                _full(wh.shape),
                                                _full(wa.shape),
                                                _full(b1.shape),
                                                _full(w2.shape),
                                                _full(b2.shape)]
    args = [h] + list(pa) + list(pb) + [c0, c1, ohn, wh, wa, b1, w2, b2]
    for a in extra:
        in_specs.append(_full(a.shape))
        args.append(a)
    out_specs = [_rows(r, 128), _full((NB, 128))]
    out_shape = [jax.ShapeDtypeStruct((n, 128), jnp.float32),
                 jax.ShapeDtypeStruct((NB, 128), jnp.float32)]
    if first:
        out_specs += [_full((NB, 128)), _rows(r, 128), _rows(r, 128)]
        out_shape += [jax.ShapeDtypeStruct((NB, 128), jnp.float32),
                      jax.ShapeDtypeStruct((n, 128), jnp.float32),
                      jax.ShapeDtypeStruct((n, 128), jnp.float32)]
    return pl.pallas_call(
        functools.partial(_node_body, first, len(pa), len(pb)),
        grid=(n // r,),
        in_specs=in_specs,
        out_specs=out_specs,
        out_shape=out_shape,
    )(*args)


def _glob_body(sumn_ref, cntn_ref, sea_ref, seb_ref, cea_ref, ceb_ref,
               virt_ref, wn_ref, we_ref, wv_ref, b1_ref, w2_ref, b2_ref,
               vnew_ref):
    nmean = sumn_ref[...] / jnp.maximum(cntn_ref[...], 1.0)
    emean = ((sea_ref[...] + seb_ref[...])
             / jnp.maximum(cea_ref[...] + ceb_ref[...], 1.0))
    hid = _silu(_dot(nmean, wn_ref[...]) + _dot(emean, we_ref[...])
                + _dot(virt_ref[...], wv_ref[...]) + b1_ref[...])
    vnew_ref[...] = virt_ref[...] + _dot(hid, w2_ref[...]) + b2_ref[...]


def _run_glob(sumn, cntn, sea, seb, cea, ceb, virt, wn, we, wv, b1, w2, b2):
    return pl.pallas_call(
        _glob_body,
        out_shape=jax.ShapeDtypeStruct((NB, 128), jnp.float32),
    )(sumn, cntn, sea, seb, cea, ceb, virt, wn, we, wv, b1, w2, b2)


def _out_body(h_ref, pa_ref, pb_ref, pc_ref, pd_ref, c0_ref, c1_ref, oh_ref,
              virt_ref, t_ref, cond_ref, wh_ref, wa_ref, wuv_ref, wut_ref,
              wuc_ref, b1_ref, w2_ref, b2_ref, ret_ref):
    cnt = jnp.maximum(c0_ref[...] + c1_ref[...], 1.0)
    agg = (pa_ref[...] + pb_ref[...] + pc_ref[...] + pd_ref[...]) / cnt
    uproj = (_dot(virt_ref[...], wuv_ref[...])
             + _dot(cond_ref[...], wuc_ref[...])
             + _dot(t_ref[...], wut_ref[...]))
    pre = (_dot(h_ref[...], wh_ref[...]) + _dot(agg, wa_ref[...])
           + _dot(oh_ref[...], uproj) + b1_ref[...])
    ret_ref[...] = _dot(_silu(pre), w2_ref[...]) + b2_ref[...]


def _run_out(h, parts, c0, c1, ohn, virt, t, cond, wh, wa, wuv, wut, wuc, b1,
             w2, b2):
    n = h.shape[0]
    r = 2000
    dout = b2.shape[1]
    in_specs = ([_rows(r, 128)] + [_rows(r, 128)] * 4
                + [_rows(r, 128), _rows(r, 128), _rows(r, NB)]
                + [_full(a.shape) for a in
                   (virt, t, cond, wh, wa, wuv, wut, wuc, b1, w2, b2)])
    return pl.pallas_call(
        _out_body,
        grid=(n // r,),
        in_specs=in_specs,
        out_specs=_rows(r, dout),
        out_shape=jax.ShapeDtypeStruct((n, dout), jnp.float32),
    )(h, *parts, c0, c1, ohn, virt, t, cond, wh, wa, wuv, wut, wuc, b1, w2, b2)


def _mesh():
    return plsc.VectorSubcoreMesh(core_axis_name="c", subcore_axis_name="s")


def _sc_gather(src, dst, tables, idx_sel, chunk0=0, n_chunks=None):
    e = src.shape[0]
    nt = len(tables)
    if n_chunks is None:
        n_chunks = e // CHUNK
    per_w = n_chunks // NW
    rem = n_chunks - per_w * NW
    assert per_w >= 3
    out_type = tuple(jax.ShapeDtypeStruct((e, t.shape[1]), jnp.float32)
                     for t in tables)
    scratch = ([pltpu.VMEM((CHUNK,), jnp.int32)] * 4
               + [pltpu.VMEM((CHUNK, t.shape[1]), jnp.float32)
                  for t in tables for _ in range(2)]
               + [pltpu.SemaphoreType.DMA] * 6)

    @functools.partial(pl.kernel, out_type=out_type, mesh=_mesh(),
                       scratch_types=scratch)
    def k(src_hbm, dst_hbm, *refs):
        tabs = refs[:nt]
        outs = refs[nt:2 * nt]
        p = 2 * nt
        isrc = refs[p:p + 2]
        idst = refs[p + 2:p + 4]
        bufs = [refs[p + 4 + 2 * t:p + 6 + 2 * t] for t in range(nt)]
        semi = refs[p + 4 + 2 * nt:p + 6 + 2 * nt]
        semg = refs[p + 6 + 2 * nt:p + 8 + 2 * nt]
        semw = refs[p + 8 + 2 * nt:p + 10 + 2 * nt]
        cid = lax.axis_index("c")
        sid = lax.axis_index("s")
        wid = sid * NC + cid

        def off_of(i):
            return pl.multiple_of((chunk0 + i * NW) * CHUNK + wid * CHUNK,
                                  CHUNK)

        def issue_idx(i, b):
            off = off_of(i)
            pltpu.async_copy(src_hbm.at[pl.ds(off, CHUNK)], isrc[b], semi[b])
            pltpu.async_copy(dst_hbm.at[pl.ds(off, CHUNK)], idst[b], semi[b])

        def wait_idx(b):
            pltpu.make_async_copy(src_hbm.at[pl.ds(0, CHUNK)], isrc[b],
                                  semi[b]).wait()
            pltpu.make_async_copy(dst_hbm.at[pl.ds(0, CHUNK)], idst[b],
                                  semi[b]).wait()

        def run_gather(b):
            descs = []
            for t in range(nt):
                ib = isrc[b] if idx_sel[t] == 0 else idst[b]
                descs.append(pltpu.async_copy(tabs[t].at[ib], bufs[t][b],
                                              semg[b]))
            for d in descs:
                d.wait()

        def issue_wb(i, b):
            off = off_of(i)
            for t in range(nt):
                pltpu.async_copy(bufs[t][b], outs[t].at[pl.ds(off, CHUNK)],
                                 semw[b])

        def wait_wb(b):
            for t in range(nt):
                pltpu.make_async_copy(bufs[t][b],
                                      outs[t].at[pl.ds(0, CHUNK)],
                                      semw[b]).wait()

        issue_idx(0, 0)

        def body(j2, carry):
            for b in range(2):
                i = 2 * j2 + b

                @pl.when(i + 1 < per_w)
                def _():
                    issue_idx(i + 1, 1 - b)

                wait_idx(b)

                @pl.when(i >= 2)
                def _():
                    wait_wb(b)

                run_gather(b)
                issue_wb(i, b)
            return carry

        lax.fori_loop(0, per_w // 2, body, 0)
        if per_w % 2:
            i = per_w - 1
            b = i % 2
            wait_idx(b)
            wait_wb(b)
            run_gather(b)
            issue_wb(i, b)
        wait_wb(0)
        wait_wb(1)
        if rem:
            @pl.when(wid < rem)
            def _():
                off = pl.multiple_of((chunk0 + per_w * NW) * CHUNK
                                     + wid * CHUNK, CHUNK)
                pltpu.sync_copy(src_hbm.at[pl.ds(off, CHUNK)], isrc[0])
                pltpu.sync_copy(dst_hbm.at[pl.ds(off, CHUNK)], idst[0])
                run_gather(0)
                for t in range(nt):
                    pltpu.sync_copy(bufs[t][0], outs[t].at[pl.ds(off, CHUNK)])

    return k(src, dst, *tables)


def _sc_scatter(v, idx, zeros_big, chunk0=0, n_chunks=None):
    e, w = v.shape
    n = zeros_big.shape[0]
    if n_chunks is None:
        n_chunks = e // CHUNK
    per_w = n_chunks // NW
    rem = n_chunks - per_w * NW
    rows = (n // NS) // 8 * 8
    tail = n - NS * rows
    out_type = tuple(jax.ShapeDtypeStruct((n, w), jnp.float32)
                     for _ in range(NC))
    assert per_w >= 3
    scratch = [pltpu.VMEM((CHUNK,), jnp.int32)] * 2 + \
              [pltpu.VMEM((CHUNK, w), jnp.float32)] * 2 + \
              [pltpu.VMEM_SHARED((n, w), jnp.float32)] + \
              [pltpu.SemaphoreType.DMA] * 4

    @functools.partial(pl.kernel, out_type=out_type, mesh=_mesh(),
                       scratch_types=scratch)
    def k(v_hbm, idx_hbm, z_hbm, out0, out1, ibuf0, ibuf1, vbuf0, vbuf1, acc,
          seml0, seml1, sema0, sema1):
        ibuf = (ibuf0, ibuf1)
        vbuf = (vbuf0, vbuf1)
        seml = (seml0, seml1)
        sema = (sema0, sema1)
        cid = lax.axis_index("c")
        sid = lax.axis_index("s")
        wid = sid * NC + cid

        @pl.when(sid == 0)
        def _():
            pltpu.sync_copy(z_hbm, acc)

        plsc.subcore_barrier()

        def off_of(i):
            return pl.multiple_of((chunk0 + i * NW) * CHUNK + wid * CHUNK,
                                  CHUNK)

        def issue_load(i, b):
            off = off_of(i)
            pltpu.async_copy(idx_hbm.at[pl.ds(off, CHUNK)], ibuf[b], seml[b])
            pltpu.async_copy(v_hbm.at[pl.ds(off, CHUNK)], vbuf[b], seml[b])

        def wait_load(b):
            pltpu.make_async_copy(idx_hbm.at[pl.ds(0, CHUNK)], ibuf[b],
                                  seml[b]).wait()
            pltpu.make_async_copy(v_hbm.at[pl.ds(0, CHUNK)], vbuf[b],
                                  seml[b]).wait()

        issue_load(0, 0)

        def body(j2, carry):
            for b in range(2):
                i = 2 * j2 + b

                @pl.when(i + 1 < per_w)
                def _():
                    issue_load(i + 1, 1 - b)

                wait_load(b)
                pltpu.sync_copy(vbuf[b], acc.at[ibuf[b]], add=True)
            return carry

        lax.fori_loop(0, per_w // 2, body, 0)
        if per_w % 2:
            b = (per_w - 1) % 2
            wait_load(b)
            pltpu.sync_copy(vbuf[b], acc.at[ibuf[b]], add=True)
        if rem:
            @pl.when(wid < rem)
            def _():
                off = pl.multiple_of((chunk0 + per_w * NW) * CHUNK
                                     + wid * CHUNK, CHUNK)
                pltpu.sync_copy(idx_hbm.at[pl.ds(off, CHUNK)], ibuf[0])
                pltpu.sync_copy(v_hbm.at[pl.ds(off, CHUNK)], vbuf[0])
                pltpu.sync_copy(vbuf[0], acc.at[ibuf[0]], add=True)
        plsc.subcore_barrier()

        r0 = pl.multiple_of(sid * rows, 8)

        @pl.when(cid == 0)
        def _():
            pltpu.sync_copy(acc.at[pl.ds(r0, rows)], out0.at[pl.ds(r0, rows)])

            @pl.when(sid == 0)
            def _():
                if tail:
                    pltpu.sync_copy(acc.at[pl.ds(NS * rows, tail)],
                                    out0.at[pl.ds(NS * rows, tail)])

        @pl.when(cid == 1)
        def _():
            pltpu.sync_copy(acc.at[pl.ds(r0, rows)], out1.at[pl.ds(r0, rows)])

            @pl.when(sid == 0)
            def _():
                if tail:
                    pltpu.sync_copy(acc.at[pl.ds(NS * rows, tail)],
                                    out1.at[pl.ds(NS * rows, tail)])

    return k(v, idx, zeros_big)


def _sc_counts(idx, zeros_big, ones_chunk):
    e = idx.shape[0]
    n, w = zeros_big.shape
    n_chunks = e // CHUNK
    per_w = n_chunks // NW
    rem = n_chunks - per_w * NW
    rows = (n // NS) // 8 * 8
    tail = n - NS * rows
    out_type = tuple(jax.ShapeDtypeStruct((n, w), jnp.float32)
                     for _ in range(NC))
    assert per_w % 2 == 0
    scratch = [pltpu.VMEM((CHUNK,), jnp.int32)] * 2 + \
              [pltpu.VMEM((CHUNK, w), jnp.float32)] + \
              [pltpu.VMEM_SHARED((n, w), jnp.float32)] + \
              [pltpu.SemaphoreType.DMA] * 4

    @functools.partial(pl.kernel, out_type=out_type, mesh=_mesh(),
                       scratch_types=scratch)
    def k(idx_hbm, z_hbm, ones_hbm, out0, out1, ibuf0, ibuf1, obuf, acc,
          seml0, seml1, sema0, sema1):
        ibuf = (ibuf0, ibuf1)
        seml = (seml0, seml1)
        sema = (sema0, sema1)
        cid = lax.axis_index("c")
        sid = lax.axis_index("s")
        wid = sid * NC + cid

        @pl.when(sid == 0)
        def _():
            pltpu.sync_copy(z_hbm, acc)

        pltpu.sync_copy(ones_hbm, obuf)
        plsc.subcore_barrier()

        def off_of(i):
            return pl.multiple_of((i * NW + wid) * CHUNK, CHUNK)

        def issue_load(i, b):
            pltpu.async_copy(idx_hbm.at[pl.ds(off_of(i), CHUNK)], ibuf[b],
                             seml[b])

        def wait_load(b):
            pltpu.make_async_copy(idx_hbm.at[pl.ds(0, CHUNK)], ibuf[b],
                                  seml[b]).wait()

        issue_load(0, 0)

        def body(j2, carry):
            for b in range(2):
                i = 2 * j2 + b

                @pl.when(i + 1 < per_w)
                def _():
                    issue_load(i + 1, 1 - b)

                wait_load(b)
                pltpu.sync_copy(obuf, acc.at[ibuf[b]], add=True)
            return carry

        lax.fori_loop(0, per_w // 2, body, 0)
        if rem:
            @pl.when(wid < rem)
            def _():
                off = pl.multiple_of((per_w * NW + wid) * CHUNK, CHUNK)
                pltpu.sync_copy(idx_hbm.at[pl.ds(off, CHUNK)], ibuf[0])
                pltpu.sync_copy(obuf, acc.at[ibuf[0]], add=True)
        plsc.subcore_barrier()

        r0 = pl.multiple_of(sid * rows, 8)

        @pl.when(cid == 0)
        def _():
            pltpu.sync_copy(acc.at[pl.ds(r0, rows)], out0.at[pl.ds(r0, rows)])

            @pl.when(sid == 0)
            def _():
                if tail:
                    pltpu.sync_copy(acc.at[pl.ds(NS * rows, tail)],
                                    out0.at[pl.ds(NS * rows, tail)])

        @pl.when(cid == 1)
        def _():
            pltpu.sync_copy(acc.at[pl.ds(r0, rows)], out1.at[pl.ds(r0, rows)])

            @pl.when(sid == 0)
            def _():
                if tail:
                    pltpu.sync_copy(acc.at[pl.ds(NS * rows, tail)],
                                    out1.at[pl.ds(NS * rows, tail)])

    return k(idx, zeros_big, ones_chunk)


def kernel(nodes, x, edges, virtual, edge_index, node_batch, lengths, t, cond,
           params):
    n = nodes.shape[0]
    src = edge_index[0]
    dst = edge_index[1]

    (w1p, b1p), (w2p, b2p) = params["pe"]
    wn_p = w1p[:128]
    wx_p = jnp.pad(w1p[128:], ((0, 128 - (w1p.shape[0] - 128)), (0, 0)))
    xp = jnp.pad(x, ((0, 0), (0, 128 - x.shape[1])))
    nb2 = node_batch.reshape(n, 1)

    def lin(layer):
        w, b = layer
        return w, b.reshape(1, -1)

    e0w1, e0b1 = lin(params["mpnn0"]["edge"][0])
    e0w2, e0b2 = lin(params["mpnn0"]["edge"][1])
    n0w1, n0b1 = lin(params["mpnn0"]["node"][0])
    n0w2, n0b2 = lin(params["mpnn0"]["node"][1])
    g0w1, g0b1 = lin(params["mpnn0"]["glob"][0])
    g0w2, g0b2 = lin(params["mpnn0"]["glob"][1])
    e1w1, e1b1 = lin(params["mpnn1"]["edge"][0])
    e1w2, e1b2 = lin(params["mpnn1"]["edge"][1])
    n1w1, n1b1 = lin(params["mpnn1"]["node"][0])
    n1w2, n1b2 = lin(params["mpnn1"]["node"][1])
    g1w1, g1b1 = lin(params["mpnn1"]["glob"][0])
    g1w2, g1b2 = lin(params["mpnn1"]["glob"][1])
    ow1, ob1 = lin(params["out"][0])
    ow2, ob2 = lin(params["out"][1])

    zeros_big = jnp.zeros((n, 128), jnp.float32)
    ones_chunk = jnp.ones((CHUNK, 128), jnp.float32)

    h0, ts0, td0, ohn = _run_pre(
        nodes, xp, nb2, wn_p, wx_p, b1p.reshape(1, -1), w2p,
        b2p.reshape(1, -1), e0w1[0:128], e0w1[128:256], e0b1)

    c0, c1 = _sc_counts(dst, zeros_big, ones_chunk)
    si0, si1 = _sc_scatter(edges, dst, zeros_big)

    e = edges.shape[0]
    hc = (e // CHUNK) // 2

    gsA, gdA = _sc_gather(src, dst, [ts0, td0], [0, 1], 0, hc)
    gsB, gdB = _sc_gather(src, dst, [ts0, td0], [0, 1], hc, hc)
    e1A, seA, ceA = _run_edge(False, gsA, gdA, edges, e0w1[256:384], e0w2,
                              e0b2, part=0, nparts=2)
    e1B, seB, ceB = _run_edge(False, gsB, gdB, edges, e0w1[256:384], e0w2,
                              e0b2, part=1, nparts=2)
    t1A0, t1A1 = _sc_scatter(e1A, dst, zeros_big, 0, hc)
    t1B0, t1B1 = _sc_scatter(e1B, dst, zeros_big, hc, hc)
    h1, sum_n0, cnt_n, ts1, td1 = _run_node(
        True, h0, [t1A0, t1A1, t1B0, t1B1], [si0, si1], c0, c1, ohn,
        n0w1[0:128], n0w1[128:256], n0b1, n0w2, n0b2,
        [e1w1[0:128], e1w1[128:256], e1b1])
    virtual1 = _run_glob(sum_n0, cnt_n, seA, seB, ceA, ceB, virtual,
                         g0w1[0:128], g0w1[128:256], g0w1[256:384], g0b1,
                         g0w2, g0b2)

    hsA, hdA = _sc_gather(src, dst, [ts1, td1], [0, 1], 0, hc)
    hsB, hdB = _sc_gather(src, dst, [ts1, td1], [0, 1], hc, hc)
    e2A, sfA = _run_edge(True, hsA, hdA, e1A, e1w1[256:384], e1w2, e1b2,
                         wv=e1w1[384:512], virt=virtual1, part=0, nparts=2)
    e2B, sfB = _run_edge(True, hsB, hdB, e1B, e1w1[256:384], e1w2, e1b2,
                         wv=e1w1[384:512], virt=virtual1, part=1, nparts=2)
    t2A0, t2A1 = _sc_scatter(e2A, dst, zeros_big, 0, hc)
    t2B0, t2B1 = _sc_scatter(e2B, dst, zeros_big, hc, hc)
    h2, sum_n1 = _run_node(
        False, h1, [t2A0, t2A1, t2B0, t2B1], [t1A0, t1A1, t1B0, t1B1],
        c0, c1, ohn, n1w1[0:128], n1w1[128:256], n1b1, n1w2, n1b2,
        [n1w1[256:384], virtual1])
    virtual2 = _run_glob(sum_n1, cnt_n, sfA, sfB, ceA, ceB, virtual1,
                         g1w1[0:128], g1w1[128:256], g1w1[256:384], g1b1,
                         g1w2, g1b2)

    ret = _run_out(h2, [t2A0, t2A1, t2B0, t2B1], c0, c1, ohn, virtual2,
                   t, cond, ow1[0:128], ow1[128:256], ow1[256:384],
                   ow1[384:512], ow1[512:576], ob1, ow2, ob2)

    edges2 = jnp.concatenate([e2A[:e // 2], e2B[e // 2:]], axis=0)
    return ((h2, edges2, virtual2, edge_index, node_batch, lengths, t, cond),
            ret)


# speedup vs baseline: 1.1665x; 1.1665x over previous
import functools

import jax
import jax.numpy as jnp
from jax import lax
from jax.experimental import pallas as pl
from jax.experimental.pallas import tpu as pltpu
from jax.experimental.pallas import tpu_sc as plsc

NB = 16
NC, NS = 2, 16
NW = NC * NS
CHUNK = 128


def _silu(x):
    return x * jax.nn.sigmoid(x)


def _dot(a, b):
    return jnp.dot(a, b, preferred_element_type=jnp.float32)


def _dotT(a, b):
    return lax.dot_general(a, b, (((0,), (0,)), ((), ())),
                           preferred_element_type=jnp.float32)


def _full(shape):
    return pl.BlockSpec(shape, lambda i: (0,) * len(shape))


def _rows(r, c):
    return pl.BlockSpec((r, c), lambda i: (i, 0))


def _pre_body(nodes_ref, xp_ref, nb_ref, wn_ref, wx_ref, b1_ref, w2_ref,
              b2_ref, ws_ref, wd_ref, be_ref, h_ref, ts_ref, td_ref, oh_ref):
    r = nodes_ref.shape[0]
    hmid = _silu(_dot(nodes_ref[...], wn_ref[...])
                 + _dot(xp_ref[...], wx_ref[...]) + b1_ref[...])
    h = _dot(hmid, w2_ref[...]) + b2_ref[...]
    h_ref[...] = h
    cols = lax.broadcasted_iota(jnp.int32, (r, NB), 1)
    oh = (nb_ref[...] == cols).astype(jnp.float32)
    oh_ref[...] = oh
    z = jnp.zeros((r, 128 - 64 - NB), jnp.float32)
    ts_ref[...] = jnp.concatenate(
        [_dot(h, ws_ref[...]) + be_ref[...], oh, z], axis=1)
    td_ref[...] = jnp.concatenate(
        [_dot(h, wd_ref[...]), jnp.zeros((r, 64), jnp.float32)], axis=1)


def _run_pre(nodes, xp, nb2, wn, wx, b1, w2, b2, ws, wd, be):
    n = nodes.shape[0]
    r = 2000
    return pl.pallas_call(
        _pre_body,
        grid=(n // r,),
        in_specs=[_rows(r, 128), _rows(r, 128), _rows(r, 1),
                  _full(wn.shape), _full(wx.shape), _full(b1.shape),
                  _full(w2.shape), _full(b2.shape), _full(ws.shape),
                  _full(wd.shape), _full(be.shape)],
        out_specs=[_rows(r, 128), _rows(r, 128), _rows(r, 128), _rows(r, NB)],
        out_shape=[jax.ShapeDtypeStruct((n, 128), jnp.float32),
                   jax.ShapeDtypeStruct((n, 128), jnp.float32),
                   jax.ShapeDtypeStruct((n, 128), jnp.float32),
                   jax.ShapeDtypeStruct((n, NB), jnp.float32)],
    )(nodes, xp, nb2, wn, wx, b1, w2, b2, ws, wd, be)


def _edge_body(megnet, has_carry, gs_ref, gd_ref, edges_ref, we_ref, w2_ref,
               b2_ref, *refs):
    if megnet:
        wv_ref, virt_ref = refs[0], refs[1]
        refs = refs[2 + (1 if has_carry else 0):]
        enew_ref, sum_eb_ref = refs[0], refs[1]
    else:
        refs = refs[(1 if has_carry else 0):]
        enew_ref, sum_eb_ref, cnt_eb_ref = refs[:3]
    gs = gs_ref[...]
    oh = gs[:, 64:64 + NB]
    pre = gs[:, 0:64] + gd_ref[...][:, 0:64] + _dot(edges_ref[...], we_ref[...])
    if megnet:
        pre = pre + _dot(oh, _dot(virt_ref[...], wv_ref[...]))
    eo = _dot(_silu(pre), w2_ref[...]) + b2_ref[...]
    enew_ref[...] = edges_ref[...] + eo

    @pl.when(pl.program_id(0) == 0)
    def _():
        sum_eb_ref[...] = jnp.zeros_like(sum_eb_ref)
        if not megnet:
            cnt_eb_ref[...] = jnp.zeros_like(cnt_eb_ref)

    sum_eb_ref[...] += _dotT(oh, eo)
    if not megnet:
        cnt_eb_ref[...] += _dotT(oh, jnp.ones_like(eo))


def _run_edge(megnet, gs, gd, edges, we, w2, b2, wv=None, virt=None, part=0,
              nparts=1, carry=None):
    e = edges.shape[0]
    r = 8000
    grid = e // nparts // r
    off = part * grid
    rows = pl.BlockSpec((r, 128), lambda i: (i + off, 0))
    in_specs = [rows, rows, rows,
                _full(we.shape), _full(w2.shape), _full(b2.shape)]
    args = [gs, gd, edges, we, w2, b2]
    if megnet:
        in_specs += [_full(wv.shape), _full(virt.shape)]
        args += [wv, virt]
    out_specs = [rows, _full((NB, 128))]
    out_shape = [jax.ShapeDtypeStruct((e, 128), jnp.float32),
                 jax.ShapeDtypeStruct((NB, 128), jnp.float32)]
    if not megnet:
        out_specs.append(_full((NB, 128)))
        out_shape.append(jax.ShapeDtypeStruct((NB, 128), jnp.float32))
    aliases = {}
    if carry is not None:
        in_specs.append(pl.BlockSpec((8, 128), lambda i: (0, 0)))
        args.append(carry)
        aliases = {len(args) - 1: 0}
    return pl.pallas_call(
        functools.partial(_edge_body, megnet, carry is not None),
        grid=(grid,),
        in_specs=in_specs,
        out_specs=out_specs,
        out_shape=out_shape,
        input_output_aliases=aliases,
    )(*args)


def _node_body(first, na, nb, h_ref, *refs):
    pa = refs[:na]
    pb = refs[na:na + nb]
    (c0_ref, c1_ref, oh_ref, wh_ref, wa_ref, b1_ref, w2_ref,
     b2_ref) = refs[na + nb:na + nb + 8]
    refs = refs[na + nb + 8:]
    if first:
        wsn_ref, wdn_ref, ben_ref = refs[0], refs[1], refs[2]
        hnew_ref, sumn_ref, cntn_ref, ts_ref, td_ref = refs[3:]
    else:
        wv_ref, virt_ref = refs[0], refs[1]
        hnew_ref, sumn_ref = refs[2], refs[3]
    oh = oh_ref[...]
    cnt = jnp.maximum(c0_ref[...] + c1_ref[...], 1.0)
    sa = sum(p[...] for p in pa[1:]) + pa[0][...]
    sb = sum(p[...] for p in pb[1:]) + pb[0][...]
    agg = (sa - sb) / cnt
    pre = _dot(h_ref[...], wh_ref[...]) + _dot(agg, wa_ref[...]) + b1_ref[...]
    if not first:
        pre = pre + _dot(oh, _dot(virt_ref[...], wv_ref[...]))
    nout = _dot(_silu(pre), w2_ref[...]) + b2_ref[...]
    hnew = h_ref[...] + nout
    hnew_ref[...] = hnew

    @pl.when(pl.program_id(0) == 0)
    def _():
        sumn_ref[...] = jnp.zeros_like(sumn_ref)
        if first:
            cntn_ref[...] = jnp.zeros_like(cntn_ref)

    sumn_ref[...] += _dotT(oh, nout)
    if first:
        cntn_ref[...] += _dotT(oh, jnp.ones_like(nout))
        r = oh.shape[0]
        z = jnp.zeros((r, 128 - 64 - NB), jnp.float32)
        ts_ref[...] = jnp.concatenate(
            [_dot(hnew, wsn_ref[...]) + ben_ref[...], oh, z], axis=1)
        td_ref[...] = jnp.concatenate(
            [_dot(hnew, wdn_ref[...]), jnp.zeros((r, 64), jnp.float32)],
            axis=1)


def _run_node(first, h, pa, pb, c0, c1, ohn, wh, wa, b1, w2, b2, extra):
    n = h.shape[0]
    r = 2000
    npart = len(pa) + len(pb)
    in_specs = [_rows(r, 128)] * (3 + npart) + [_rows(r, NB),
                                                _full(wh.shape),
                                                _full(wa.shape),
                                                _full(b1.shape),
                                                _full(w2.shape),
                                                _full(b2.shape)]
    args = [h] + list(pa) + list(pb) + [c0, c1, ohn, wh, wa, b1, w2, b2]
    for a in extra:
        in_specs.append(_full(a.shape))
        args.append(a)
    out_specs = [_rows(r, 128), _full((NB, 128))]
    out_shape = [jax.ShapeDtypeStruct((n, 128), jnp.float32),
                 jax.ShapeDtypeStruct((NB, 128), jnp.float32)]
    if first:
        out_specs += [_full((NB, 128)), _rows(r, 128), _rows(r, 128)]
        out_shape += [jax.ShapeDtypeStruct((NB, 128), jnp.float32),
                      jax.ShapeDtypeStruct((n, 128), jnp.float32),
                      jax.ShapeDtypeStruct((n, 128), jnp.float32)]
    return pl.pallas_call(
        functools.partial(_node_body, first, len(pa), len(pb)),
        grid=(n // r,),
        in_specs=in_specs,
        out_specs=out_specs,
        out_shape=out_shape,
    )(*args)


def _glob_body(sumn_ref, cntn_ref, sea_ref, seb_ref, cea_ref, ceb_ref,
               virt_ref, wn_ref, we_ref, wv_ref, b1_ref, w2_ref, b2_ref,
               vnew_ref):
    nmean = sumn_ref[...] / jnp.maximum(cntn_ref[...], 1.0)
    emean = ((sea_ref[...] + seb_ref[...])
             / jnp.maximum(cea_ref[...] + ceb_ref[...], 1.0))
    hid = _silu(_dot(nmean, wn_ref[...]) + _dot(emean, we_ref[...])
                + _dot(virt_ref[...], wv_ref[...]) + b1_ref[...])
    vnew_ref[...] = virt_ref[...] + _dot(hid, w2_ref[...]) + b2_ref[...]


def _run_glob(sumn, cntn, sea, seb, cea, ceb, virt, wn, we, wv, b1, w2, b2):
    return pl.pallas_call(
        _glob_body,
        out_shape=jax.ShapeDtypeStruct((NB, 128), jnp.float32),
    )(sumn, cntn, sea, seb, cea, ceb, virt, wn, we, wv, b1, w2, b2)


def _out_body(h_ref, pa_ref, pb_ref, c0_ref, c1_ref, oh_ref,
              virt_ref, t_ref, cond_ref, wh_ref, wa_ref, wuv_ref, wut_ref,
              wuc_ref, b1_ref, w2_ref, b2_ref, ret_ref):
    cnt = jnp.maximum(c0_ref[...] + c1_ref[...], 1.0)
    agg = (pa_ref[...] + pb_ref[...]) / cnt
    uproj = (_dot(virt_ref[...], wuv_ref[...])
             + _dot(cond_ref[...], wuc_ref[...])
             + _dot(t_ref[...], wut_ref[...]))
    pre = (_dot(h_ref[...], wh_ref[...]) + _dot(agg, wa_ref[...])
           + _dot(oh_ref[...], uproj) + b1_ref[...])
    ret_ref[...] = _dot(_silu(pre), w2_ref[...]) + b2_ref[...]


def _run_out(h, parts, c0, c1, ohn, virt, t, cond, wh, wa, wuv, wut, wuc, b1,
             w2, b2):
    n = h.shape[0]
    r = 2000
    dout = b2.shape[1]
    in_specs = ([_rows(r, 128)] + [_rows(r, 128)] * 2
                + [_rows(r, 128), _rows(r, 128), _rows(r, NB)]
                + [_full(a.shape) for a in
                   (virt, t, cond, wh, wa, wuv, wut, wuc, b1, w2, b2)])
    return pl.pallas_call(
        _out_body,
        grid=(n // r,),
        in_specs=in_specs,
        out_specs=_rows(r, dout),
        out_shape=jax.ShapeDtypeStruct((n, dout), jnp.float32),
    )(h, *parts, c0, c1, ohn, virt, t, cond, wh, wa, wuv, wut, wuc, b1, w2, b2)


def _mesh():
    return plsc.VectorSubcoreMesh(core_axis_name="c", subcore_axis_name="s")


def _sc_gather(src, dst, tables, idx_sel, chunk0=0, n_chunks=None):
    e = src.shape[0]
    nt = len(tables)
    if n_chunks is None:
        n_chunks = e // CHUNK
    per_w = n_chunks // NW
    rem = n_chunks - per_w * NW
    assert per_w >= 3
    out_type = tuple(jax.ShapeDtypeStruct((e, t.shape[1]), jnp.float32)
                     for t in tables)
    scratch = ([pltpu.VMEM((CHUNK,), jnp.int32)] * 4
               + [pltpu.VMEM((CHUNK, t.shape[1]), jnp.float32)
                  for t in tables for _ in range(2)]
               + [pltpu.SemaphoreType.DMA] * 6)

    @functools.partial(pl.kernel, out_type=out_type, mesh=_mesh(),
                       scratch_types=scratch)
    def k(src_hbm, dst_hbm, *refs):
        tabs = refs[:nt]
        outs = refs[nt:2 * nt]
        p = 2 * nt
        isrc = refs[p:p + 2]
        idst = refs[p + 2:p + 4]
        bufs = [refs[p + 4 + 2 * t:p + 6 + 2 * t] for t in range(nt)]
        semi = refs[p + 4 + 2 * nt:p + 6 + 2 * nt]
        semg = refs[p + 6 + 2 * nt:p + 8 + 2 * nt]
        semw = refs[p + 8 + 2 * nt:p + 10 + 2 * nt]
        cid = lax.axis_index("c")
        sid = lax.axis_index("s")
        wid = sid * NC + cid

        def off_of(i):
            return pl.multiple_of((chunk0 + i * NW) * CHUNK + wid * CHUNK,
                                  CHUNK)

        def issue_idx(i, b):
            off = off_of(i)
            pltpu.async_copy(src_hbm.at[pl.ds(off, CHUNK)], isrc[b], semi[b])
            pltpu.async_copy(dst_hbm.at[pl.ds(off, CHUNK)], idst[b], semi[b])

        def wait_idx(b):
            pltpu.make_async_copy(src_hbm.at[pl.ds(0, CHUNK)], isrc[b],
                                  semi[b]).wait()
            pltpu.make_async_copy(dst_hbm.at[pl.ds(0, CHUNK)], idst[b],
                                  semi[b]).wait()

        def run_gather(b):
            descs = []
            for t in range(nt):
                ib = isrc[b] if idx_sel[t] == 0 else idst[b]
                descs.append(pltpu.async_copy(tabs[t].at[ib], bufs[t][b],
                                              semg[b]))
            for d in descs:
                d.wait()

        def issue_wb(i, b):
            off = off_of(i)
            for t in range(nt):
                pltpu.async_copy(bufs[t][b], outs[t].at[pl.ds(off, CHUNK)],
                                 semw[b])

        def wait_wb(b):
            for t in range(nt):
                pltpu.make_async_copy(bufs[t][b],
                                      outs[t].at[pl.ds(0, CHUNK)],
                                      semw[b]).wait()

        issue_idx(0, 0)

        def body(j2, carry):
            for b in range(2):
                i = 2 * j2 + b

                @pl.when(i + 1 < per_w)
                def _():
                    issue_idx(i + 1, 1 - b)

                wait_idx(b)

                @pl.when(i >= 2)
                def _():
                    wait_wb(b)

                run_gather(b)
                issue_wb(i, b)
            return carry

        lax.fori_loop(0, per_w // 2, body, 0)
        if per_w % 2:
            i = per_w - 1
            b = i % 2
            wait_idx(b)
            wait_wb(b)
            run_gather(b)
            issue_wb(i, b)
        wait_wb(0)
        wait_wb(1)
        if rem:
            @pl.when(wid < rem)
            def _():
                off = pl.multiple_of((chunk0 + per_w * NW) * CHUNK
                                     + wid * CHUNK, CHUNK)
                pltpu.sync_copy(src_hbm.at[pl.ds(off, CHUNK)], isrc[0])
                pltpu.sync_copy(dst_hbm.at[pl.ds(off, CHUNK)], idst[0])
                run_gather(0)
                for t in range(nt):
                    pltpu.sync_copy(bufs[t][0], outs[t].at[pl.ds(off, CHUNK)])

    return k(src, dst, *tables)


def _sc_scatter(v, idx, zeros_big, chunk0=0, n_chunks=None):
    e, w = v.shape
    n = zeros_big.shape[0]
    if n_chunks is None:
        n_chunks = e // CHUNK
    per_w = n_chunks // NW
    rem = n_chunks - per_w * NW
    rows = (n // NS) // 8 * 8
    tail = n - NS * rows
    out_type = tuple(jax.ShapeDtypeStruct((n, w), jnp.float32)
                     for _ in range(NC))
    assert per_w >= 3
    scratch = [pltpu.VMEM((CHUNK,), jnp.int32)] * 2 + \
              [pltpu.VMEM((CHUNK, w), jnp.float32)] * 2 + \
              [pltpu.VMEM_SHARED((n, w), jnp.float32)] + \
              [pltpu.SemaphoreType.DMA] * 4

    @functools.partial(pl.kernel, out_type=out_type, mesh=_mesh(),
                       scratch_types=scratch)
    def k(v_hbm, idx_hbm, z_hbm, out0, out1, ibuf0, ibuf1, vbuf0, vbuf1, acc,
          seml0, seml1, sema0, sema1):
        ibuf = (ibuf0, ibuf1)
        vbuf = (vbuf0, vbuf1)
        seml = (seml0, seml1)
        sema = (sema0, sema1)
        cid = lax.axis_index("c")
        sid = lax.axis_index("s")
        wid = sid * NC + cid

        @pl.when(sid == 0)
        def _():
            pltpu.sync_copy(z_hbm, acc)

        plsc.subcore_barrier()

        def off_of(i):
            return pl.multiple_of((chunk0 + i * NW) * CHUNK + wid * CHUNK,
                                  CHUNK)

        def issue_load(i, b):
            off = off_of(i)
            pltpu.async_copy(idx_hbm.at[pl.ds(off, CHUNK)], ibuf[b], seml[b])
            pltpu.async_copy(v_hbm.at[pl.ds(off, CHUNK)], vbuf[b], seml[b])

        def wait_load(b):
            pltpu.make_async_copy(idx_hbm.at[pl.ds(0, CHUNK)], ibuf[b],
                                  seml[b]).wait()
            pltpu.make_async_copy(v_hbm.at[pl.ds(0, CHUNK)], vbuf[b],
                                  seml[b]).wait()

        issue_load(0, 0)

        def body(j2, carry):
            for b in range(2):
                i = 2 * j2 + b

                @pl.when(i + 1 < per_w)
                def _():
                    issue_load(i + 1, 1 - b)

                wait_load(b)
                pltpu.sync_copy(vbuf[b], acc.at[ibuf[b]], add=True)
            return carry

        lax.fori_loop(0, per_w // 2, body, 0)
        if per_w % 2:
            b = (per_w - 1) % 2
            wait_load(b)
            pltpu.sync_copy(vbuf[b], acc.at[ibuf[b]], add=True)
        if rem:
            @pl.when(wid < rem)
            def _():
                off = pl.multiple_of((chunk0 + per_w * NW) * CHUNK
                                     + wid * CHUNK, CHUNK)
                pltpu.sync_copy(idx_hbm.at[pl.ds(off, CHUNK)], ibuf[0])
                pltpu.sync_copy(v_hbm.at[pl.ds(off, CHUNK)], vbuf[0])
                pltpu.sync_copy(vbuf[0], acc.at[ibuf[0]], add=True)
        plsc.subcore_barrier()

        r0 = pl.multiple_of(sid * rows, 8)

        @pl.when(cid == 0)
        def _():
            pltpu.sync_copy(acc.at[pl.ds(r0, rows)], out0.at[pl.ds(r0, rows)])

            @pl.when(sid == 0)
            def _():
                if tail:
                    pltpu.sync_copy(acc.at[pl.ds(NS * rows, tail)],
                                    out0.at[pl.ds(NS * rows, tail)])

        @pl.when(cid == 1)
        def _():
            pltpu.sync_copy(acc.at[pl.ds(r0, rows)], out1.at[pl.ds(r0, rows)])

            @pl.when(sid == 0)
            def _():
                if tail:
                    pltpu.sync_copy(acc.at[pl.ds(NS * rows, tail)],
                                    out1.at[pl.ds(NS * rows, tail)])

    return k(v, idx, zeros_big)


def _sc_counts(idx, zeros_big, ones_chunk):
    e = idx.shape[0]
    n, w = zeros_big.shape
    n_chunks = e // CHUNK
    per_w = n_chunks // NW
    rem = n_chunks - per_w * NW
    rows = (n // NS) // 8 * 8
    tail = n - NS * rows
    out_type = tuple(jax.ShapeDtypeStruct((n, w), jnp.float32)
                     for _ in range(NC))
    assert per_w % 2 == 0
    scratch = [pltpu.VMEM((CHUNK,), jnp.int32)] * 2 + \
              [pltpu.VMEM((CHUNK, w), jnp.float32)] + \
              [pltpu.VMEM_SHARED((n, w), jnp.float32)] + \
              [pltpu.SemaphoreType.DMA] * 4

    @functools.partial(pl.kernel, out_type=out_type, mesh=_mesh(),
                       scratch_types=scratch)
    def k(idx_hbm, z_hbm, ones_hbm, out0, out1, ibuf0, ibuf1, obuf, acc,
          seml0, seml1, sema0, sema1):
        ibuf = (ibuf0, ibuf1)
        seml = (seml0, seml1)
        sema = (sema0, sema1)
        cid = lax.axis_index("c")
        sid = lax.axis_index("s")
        wid = sid * NC + cid

        @pl.when(sid == 0)
        def _():
            pltpu.sync_copy(z_hbm, acc)

        pltpu.sync_copy(ones_hbm, obuf)
        plsc.subcore_barrier()

        def off_of(i):
            return pl.multiple_of((i * NW + wid) * CHUNK, CHUNK)

        def issue_load(i, b):
            pltpu.async_copy(idx_hbm.at[pl.ds(off_of(i), CHUNK)], ibuf[b],
                             seml[b])

        def wait_load(b):
            pltpu.make_async_copy(idx_hbm.at[pl.ds(0, CHUNK)], ibuf[b],
                                  seml[b]).wait()

        issue_load(0, 0)

        def body(j2, carry):
            for b in range(2):
                i = 2 * j2 + b

                @pl.when(i + 1 < per_w)
                def _():
                    issue_load(i + 1, 1 - b)

                wait_load(b)
                pltpu.sync_copy(obuf, acc.at[ibuf[b]], add=True)
            return carry

        lax.fori_loop(0, per_w // 2, body, 0)
        if rem:
            @pl.when(wid < rem)
            def _():
                off = pl.multiple_of((per_w * NW + wid) * CHUNK, CHUNK)
                pltpu.sync_copy(idx_hbm.at[pl.ds(off, CHUNK)], ibuf[0])
                pltpu.sync_copy(obuf, acc.at[ibuf[0]], add=True)
        plsc.subcore_barrier()

        r0 = pl.multiple_of(sid * rows, 8)

        @pl.when(cid == 0)
        def _():
            pltpu.sync_copy(acc.at[pl.ds(r0, rows)], out0.at[pl.ds(r0, rows)])

            @pl.when(sid == 0)
            def _():
                if tail:
                    pltpu.sync_copy(acc.at[pl.ds(NS * rows, tail)],
                                    out0.at[pl.ds(NS * rows, tail)])

        @pl.when(cid == 1)
        def _():
            pltpu.sync_copy(acc.at[pl.ds(r0, rows)], out1.at[pl.ds(r0, rows)])

            @pl.when(sid == 0)
            def _():
                if tail:
                    pltpu.sync_copy(acc.at[pl.ds(NS * rows, tail)],
                                    out1.at[pl.ds(NS * rows, tail)])

    return k(idx, zeros_big, ones_chunk)


def kernel(nodes, x, edges, virtual, edge_index, node_batch, lengths, t, cond,
           params):
    n = nodes.shape[0]
    src = edge_index[0]
    dst = edge_index[1]

    (w1p, b1p), (w2p, b2p) = params["pe"]
    wn_p = w1p[:128]
    wx_p = jnp.pad(w1p[128:], ((0, 128 - (w1p.shape[0] - 128)), (0, 0)))
    xp = jnp.pad(x, ((0, 0), (0, 128 - x.shape[1])))
    nb2 = node_batch.reshape(n, 1)

    def lin(layer):
        w, b = layer
        return w, b.reshape(1, -1)

    e0w1, e0b1 = lin(params["mpnn0"]["edge"][0])
    e0w2, e0b2 = lin(params["mpnn0"]["edge"][1])
    n0w1, n0b1 = lin(params["mpnn0"]["node"][0])
    n0w2, n0b2 = lin(params["mpnn0"]["node"][1])
    g0w1, g0b1 = lin(params["mpnn0"]["glob"][0])
    g0w2, g0b2 = lin(params["mpnn0"]["glob"][1])
    e1w1, e1b1 = lin(params["mpnn1"]["edge"][0])
    e1w2, e1b2 = lin(params["mpnn1"]["edge"][1])
    n1w1, n1b1 = lin(params["mpnn1"]["node"][0])
    n1w2, n1b2 = lin(params["mpnn1"]["node"][1])
    g1w1, g1b1 = lin(params["mpnn1"]["glob"][0])
    g1w2, g1b2 = lin(params["mpnn1"]["glob"][1])
    ow1, ob1 = lin(params["out"][0])
    ow2, ob2 = lin(params["out"][1])

    zeros_big = jnp.zeros((n, 128), jnp.float32)
    ones_chunk = jnp.ones((CHUNK, 128), jnp.float32)

    h0, ts0, td0, ohn = _run_pre(
        nodes, xp, nb2, wn_p, wx_p, b1p.reshape(1, -1), w2p,
        b2p.reshape(1, -1), e0w1[0:128], e0w1[128:256], e0b1)

    c0, c1 = _sc_counts(dst, zeros_big, ones_chunk)
    si0, si1 = _sc_scatter(edges, dst, zeros_big)

    e = edges.shape[0]
    hc = (e // CHUNK) // 2

    gsA, gdA = _sc_gather(src, dst, [ts0, td0], [0, 1], 0, hc)
    gsB, gdB = _sc_gather(src, dst, [ts0, td0], [0, 1], hc, hc)
    e1A, seA, ceA = _run_edge(False, gsA, gdA, edges, e0w1[256:384], e0w2,
                              e0b2, part=0, nparts=2)
    edges1, seB, ceB = _run_edge(False, gsB, gdB, edges, e0w1[256:384], e0w2,
                                 e0b2, part=1, nparts=2, carry=e1A)
    t10, t11 = _sc_scatter(edges1, dst, zeros_big)
    h1, sum_n0, cnt_n, ts1, td1 = _run_node(
        True, h0, [t10, t11], [si0, si1], c0, c1, ohn,
        n0w1[0:128], n0w1[128:256], n0b1, n0w2, n0b2,
        [e1w1[0:128], e1w1[128:256], e1b1])
    virtual1 = _run_glob(sum_n0, cnt_n, seA, seB, ceA, ceB, virtual,
                         g0w1[0:128], g0w1[128:256], g0w1[256:384], g0b1,
                         g0w2, g0b2)

    hsA, hdA = _sc_gather(src, dst, [ts1, td1], [0, 1], 0, hc)
    hsB, hdB = _sc_gather(src, dst, [ts1, td1], [0, 1], hc, hc)
    e2A, sfA = _run_edge(True, hsA, hdA, edges1, e1w1[256:384], e1w2, e1b2,
                         wv=e1w1[384:512], virt=virtual1, part=0, nparts=2)
    edges2, sfB = _run_edge(True, hsB, hdB, edges1, e1w1[256:384], e1w2,
                            e1b2, wv=e1w1[384:512], virt=virtual1, part=1,
                            nparts=2, carry=e2A)
    t20, t21 = _sc_scatter(edges2, dst, zeros_big)
    h2, sum_n1 = _run_node(
        False, h1, [t20, t21], [t10, t11],
        c0, c1, ohn, n1w1[0:128], n1w1[128:256], n1b1, n1w2, n1b2,
        [n1w1[256:384], virtual1])
    virtual2 = _run_glob(sum_n1, cnt_n, sfA, sfB, ceA, ceB, virtual1,
                         g1w1[0:128], g1w1[128:256], g1w1[256:384], g1b1,
                         g1w2, g1b2)

    ret = _run_out(h2, [t20, t21], c0, c1, ohn, virtual2,
                   t, cond, ow1[0:128], ow1[128:256], ow1[256:384],
                   ow1[384:512], ow1[512:576], ob1, ow2, ob2)

    return ((h2, edges2, virtual2, edge_index, node_batch, lengths, t, cond),
            ret)


# speedup vs baseline: 1.1955x; 1.0249x over previous
import functools

import jax
import jax.numpy as jnp
from jax import lax
from jax.experimental import pallas as pl
from jax.experimental.pallas import tpu as pltpu
from jax.experimental.pallas import tpu_sc as plsc

NB = 16
NC, NS = 2, 16
NW = NC * NS
CHUNK = 128


def _silu(x):
    return x * jax.nn.sigmoid(x)


def _dot(a, b):
    return jnp.dot(a, b, preferred_element_type=jnp.float32)


def _dotT(a, b):
    return lax.dot_general(a, b, (((0,), (0,)), ((), ())),
                           preferred_element_type=jnp.float32)


def _full(shape):
    return pl.BlockSpec(shape, lambda i: (0,) * len(shape))


def _rows(r, c):
    return pl.BlockSpec((r, c), lambda i: (i, 0))


def _pre_body(nodes_ref, xp_ref, nb_ref, wn_ref, wx_ref, b1_ref, w2_ref,
              b2_ref, ws_ref, wd_ref, be_ref, h_ref, ts_ref, td_ref, oh_ref):
    r = nodes_ref.shape[0]
    hmid = _silu(_dot(nodes_ref[...], wn_ref[...])
                 + _dot(xp_ref[...], wx_ref[...]) + b1_ref[...])
    h = _dot(hmid, w2_ref[...]) + b2_ref[...]
    h_ref[...] = h
    cols = lax.broadcasted_iota(jnp.int32, (r, NB), 1)
    oh = (nb_ref[...] == cols).astype(jnp.float32)
    oh_ref[...] = oh
    z = jnp.zeros((r, 128 - 64 - NB), jnp.float32)
    ts_ref[...] = jnp.concatenate(
        [_dot(h, ws_ref[...]) + be_ref[...], oh, z], axis=1)
    td_ref[...] = jnp.concatenate(
        [_dot(h, wd_ref[...]), jnp.zeros((r, 64), jnp.float32)], axis=1)


def _run_pre(nodes, xp, nb2, wn, wx, b1, w2, b2, ws, wd, be):
    n = nodes.shape[0]
    r = 2000
    return pl.pallas_call(
        _pre_body,
        grid=(n // r,),
        in_specs=[_rows(r, 128), _rows(r, 128), _rows(r, 1),
                  _full(wn.shape), _full(wx.shape), _full(b1.shape),
                  _full(w2.shape), _full(b2.shape), _full(ws.shape),
                  _full(wd.shape), _full(be.shape)],
        out_specs=[_rows(r, 128), _rows(r, 128), _rows(r, 128), _rows(r, NB)],
        out_shape=[jax.ShapeDtypeStruct((n, 128), jnp.float32),
                   jax.ShapeDtypeStruct((n, 128), jnp.float32),
                   jax.ShapeDtypeStruct((n, 128), jnp.float32),
                   jax.ShapeDtypeStruct((n, NB), jnp.float32)],
    )(nodes, xp, nb2, wn, wx, b1, w2, b2, ws, wd, be)


def _edge_body(megnet, has_carry, gs_ref, gd_ref, edges_ref, we_ref, w2_ref,
               b2_ref, *refs):
    if megnet:
        wv_ref, virt_ref = refs[0], refs[1]
        refs = refs[2 + (1 if has_carry else 0):]
        enew_ref, sum_eb_ref = refs[0], refs[1]
    else:
        refs = refs[(1 if has_carry else 0):]
        enew_ref, sum_eb_ref, cnt_eb_ref = refs[:3]
    gs = gs_ref[...]
    oh = gs[:, 64:64 + NB]
    pre = gs[:, 0:64] + gd_ref[...][:, 0:64] + _dot(edges_ref[...], we_ref[...])
    if megnet:
        pre = pre + _dot(oh, _dot(virt_ref[...], wv_ref[...]))
    eo = _dot(_silu(pre), w2_ref[...]) + b2_ref[...]
    enew_ref[...] = edges_ref[...] + eo

    @pl.when(pl.program_id(0) == 0)
    def _():
        sum_eb_ref[...] = jnp.zeros_like(sum_eb_ref)
        if not megnet:
            cnt_eb_ref[...] = jnp.zeros_like(cnt_eb_ref)

    sum_eb_ref[...] += _dotT(oh, eo)
    if not megnet:
        cnt_eb_ref[...] += _dotT(oh, jnp.ones_like(eo))


def _run_edge(megnet, gs, gd, edges, we, w2, b2, wv=None, virt=None, part=0,
              nparts=1, carry=None):
    e = edges.shape[0]
    r = 8000
    grid = e // nparts // r
    off = part * grid
    rows = pl.BlockSpec((r, 128), lambda i: (i + off, 0))
    in_specs = [rows, rows, rows,
                _full(we.shape), _full(w2.shape), _full(b2.shape)]
    args = [gs, gd, edges, we, w2, b2]
    if megnet:
        in_specs += [_full(wv.shape), _full(virt.shape)]
        args += [wv, virt]
    out_specs = [rows, _full((NB, 128))]
    out_shape = [jax.ShapeDtypeStruct((e, 128), jnp.float32),
                 jax.ShapeDtypeStruct((NB, 128), jnp.float32)]
    if not megnet:
        out_specs.append(_full((NB, 128)))
        out_shape.append(jax.ShapeDtypeStruct((NB, 128), jnp.float32))
    aliases = {}
    if carry is not None:
        in_specs.append(pl.BlockSpec((8, 128), lambda i: (0, 0)))
        args.append(carry)
        aliases = {len(args) - 1: 0}
    return pl.pallas_call(
        functools.partial(_edge_body, megnet, carry is not None),
        grid=(grid,),
        in_specs=in_specs,
        out_specs=out_specs,
        out_shape=out_shape,
        input_output_aliases=aliases,
    )(*args)


def _node_body(first, na, nb, h_ref, *refs):
    pa = refs[:na]
    pb = refs[na:na + nb]
    (c0_ref, c1_ref, oh_ref, wh_ref, wa_ref, b1_ref, w2_ref,
     b2_ref) = refs[na + nb:na + nb + 8]
    refs = refs[na + nb + 8:]
    if first:
        wsn_ref, wdn_ref, ben_ref = refs[0], refs[1], refs[2]
        hnew_ref, sumn_ref, cntn_ref, ts_ref, td_ref = refs[3:]
    else:
        wv_ref, virt_ref = refs[0], refs[1]
        hnew_ref, sumn_ref = refs[2], refs[3]
    oh = oh_ref[...]
    cnt = jnp.maximum(c0_ref[...] + c1_ref[...], 1.0)
    sa = sum(p[...] for p in pa[1:]) + pa[0][...]
    sb = sum(p[...] for p in pb[1:]) + pb[0][...]
    agg = (sa - sb) / cnt
    pre = _dot(h_ref[...], wh_ref[...]) + _dot(agg, wa_ref[...]) + b1_ref[...]
    if not first:
        pre = pre + _dot(oh, _dot(virt_ref[...], wv_ref[...]))
    nout = _dot(_silu(pre), w2_ref[...]) + b2_ref[...]
    hnew = h_ref[...] + nout
    hnew_ref[...] = hnew

    @pl.when(pl.program_id(0) == 0)
    def _():
        sumn_ref[...] = jnp.zeros_like(sumn_ref)
        if first:
            cntn_ref[...] = jnp.zeros_like(cntn_ref)

    sumn_ref[...] += _dotT(oh, nout)
    if first:
        cntn_ref[...] += _dotT(oh, jnp.ones_like(nout))
        r = oh.shape[0]
        z = jnp.zeros((r, 128 - 64 - NB), jnp.float32)
        ts_ref[...] = jnp.concatenate(
            [_dot(hnew, wsn_ref[...]) + ben_ref[...], oh, z], axis=1)
        td_ref[...] = jnp.concatenate(
            [_dot(hnew, wdn_ref[...]), jnp.zeros((r, 64), jnp.float32)],
            axis=1)


def _run_node(first, h, pa, pb, c0, c1, ohn, wh, wa, b1, w2, b2, extra):
    n = h.shape[0]
    r = 2000
    npart = len(pa) + len(pb)
    in_specs = [_rows(r, 128)] * (3 + npart) + [_rows(r, NB),
                                                _full(wh.shape),
                                                _full(wa.shape),
                                                _full(b1.shape),
                                                _full(w2.shape),
                                                _full(b2.shape)]
    args = [h] + list(pa) + list(pb) + [c0, c1, ohn, wh, wa, b1, w2, b2]
    for a in extra:
        in_specs.append(_full(a.shape))
        args.append(a)
    out_specs = [_rows(r, 128), _full((NB, 128))]
    out_shape = [jax.ShapeDtypeStruct((n, 128), jnp.float32),
                 jax.ShapeDtypeStruct((NB, 128), jnp.float32)]
    if first:
        out_specs += [_full((NB, 128)), _rows(r, 128), _rows(r, 128)]
        out_shape += [jax.ShapeDtypeStruct((NB, 128), jnp.float32),
                      jax.ShapeDtypeStruct((n, 128), jnp.float32),
                      jax.ShapeDtypeStruct((n, 128), jnp.float32)]
    return pl.pallas_call(
        functools.partial(_node_body, first, len(pa), len(pb)),
        grid=(n // r,),
        in_specs=in_specs,
        out_specs=out_specs,
        out_shape=out_shape,
    )(*args)


def _glob_body(sumn_ref, cntn_ref, sume_ref, cnte_ref, virt_ref, wn_ref,
               we_ref, wv_ref, b1_ref, w2_ref, b2_ref, vnew_ref):
    nmean = sumn_ref[...] / jnp.maximum(cntn_ref[...], 1.0)
    emean = sume_ref[...] / jnp.maximum(cnte_ref[...], 1.0)
    hid = _silu(_dot(nmean, wn_ref[...]) + _dot(emean, we_ref[...])
                + _dot(virt_ref[...], wv_ref[...]) + b1_ref[...])
    vnew_ref[...] = virt_ref[...] + _dot(hid, w2_ref[...]) + b2_ref[...]


def _run_glob(sumn, cntn, sume, cnte, virt, wn, we, wv, b1, w2, b2):
    return pl.pallas_call(
        _glob_body,
        out_shape=jax.ShapeDtypeStruct((NB, 128), jnp.float32),
    )(sumn, cntn, sume, cnte, virt, wn, we, wv, b1, w2, b2)


def _out_body(h_ref, pa_ref, pb_ref, c0_ref, c1_ref, oh_ref,
              virt_ref, t_ref, cond_ref, wh_ref, wa_ref, wuv_ref, wut_ref,
              wuc_ref, b1_ref, w2_ref, b2_ref, ret_ref):
    cnt = jnp.maximum(c0_ref[...] + c1_ref[...], 1.0)
    agg = (pa_ref[...] + pb_ref[...]) / cnt
    uproj = (_dot(virt_ref[...], wuv_ref[...])
             + _dot(cond_ref[...], wuc_ref[...])
             + _dot(t_ref[...], wut_ref[...]))
    pre = (_dot(h_ref[...], wh_ref[...]) + _dot(agg, wa_ref[...])
           + _dot(oh_ref[...], uproj) + b1_ref[...])
    ret_ref[...] = _dot(_silu(pre), w2_ref[...]) + b2_ref[...]


def _run_out(h, parts, c0, c1, ohn, virt, t, cond, wh, wa, wuv, wut, wuc, b1,
             w2, b2):
    n = h.shape[0]
    r = 2000
    dout = b2.shape[1]
    in_specs = ([_rows(r, 128)] + [_rows(r, 128)] * 2
                + [_rows(r, 128), _rows(r, 128), _rows(r, NB)]
                + [_full(a.shape) for a in
                   (virt, t, cond, wh, wa, wuv, wut, wuc, b1, w2, b2)])
    return pl.pallas_call(
        _out_body,
        grid=(n // r,),
        in_specs=in_specs,
        out_specs=_rows(r, dout),
        out_shape=jax.ShapeDtypeStruct((n, dout), jnp.float32),
    )(h, *parts, c0, c1, ohn, virt, t, cond, wh, wa, wuv, wut, wuc, b1, w2, b2)


def _mesh():
    return plsc.VectorSubcoreMesh(core_axis_name="c", subcore_axis_name="s")


def _sc_gather(src, dst, tables, idx_sel, chunk0=0, n_chunks=None):
    e = src.shape[0]
    nt = len(tables)
    if n_chunks is None:
        n_chunks = e // CHUNK
    per_w = n_chunks // NW
    rem = n_chunks - per_w * NW
    assert per_w >= 3
    out_type = tuple(jax.ShapeDtypeStruct((e, t.shape[1]), jnp.float32)
                     for t in tables)
    scratch = ([pltpu.VMEM((CHUNK,), jnp.int32)] * 4
               + [pltpu.VMEM((CHUNK, t.shape[1]), jnp.float32)
                  for t in tables for _ in range(2)]
               + [pltpu.SemaphoreType.DMA] * 6)

    @functools.partial(pl.kernel, out_type=out_type, mesh=_mesh(),
                       scratch_types=scratch)
    def k(src_hbm, dst_hbm, *refs):
        tabs = refs[:nt]
        outs = refs[nt:2 * nt]
        p = 2 * nt
        isrc = refs[p:p + 2]
        idst = refs[p + 2:p + 4]
        bufs = [refs[p + 4 + 2 * t:p + 6 + 2 * t] for t in range(nt)]
        semi = refs[p + 4 + 2 * nt:p + 6 + 2 * nt]
        semg = refs[p + 6 + 2 * nt:p + 8 + 2 * nt]
        semw = refs[p + 8 + 2 * nt:p + 10 + 2 * nt]
        cid = lax.axis_index("c")
        sid = lax.axis_index("s")
        wid = sid * NC + cid

        def off_of(i):
            return pl.multiple_of((chunk0 + i * NW) * CHUNK + wid * CHUNK,
                                  CHUNK)

        def issue_idx(i, b):
            off = off_of(i)
            pltpu.async_copy(src_hbm.at[pl.ds(off, CHUNK)], isrc[b], semi[b])
            pltpu.async_copy(dst_hbm.at[pl.ds(off, CHUNK)], idst[b], semi[b])

        def wait_idx(b):
            pltpu.make_async_copy(src_hbm.at[pl.ds(0, CHUNK)], isrc[b],
                                  semi[b]).wait()
            pltpu.make_async_copy(dst_hbm.at[pl.ds(0, CHUNK)], idst[b],
                                  semi[b]).wait()

        def run_gather(b):
            descs = []
            for t in range(nt):
                ib = isrc[b] if idx_sel[t] == 0 else idst[b]
                descs.append(pltpu.async_copy(tabs[t].at[ib], bufs[t][b],
                                              semg[b]))
            for d in descs:
                d.wait()

        def issue_wb(i, b):
            off = off_of(i)
            for t in range(nt):
                pltpu.async_copy(bufs[t][b], outs[t].at[pl.ds(off, CHUNK)],
                                 semw[b])

        def wait_wb(b):
            for t in range(nt):
                pltpu.make_async_copy(bufs[t][b],
                                      outs[t].at[pl.ds(0, CHUNK)],
                                      semw[b]).wait()

        issue_idx(0, 0)

        def body(j2, carry):
            for b in range(2):
                i = 2 * j2 + b

                @pl.when(i + 1 < per_w)
                def _():
                    issue_idx(i + 1, 1 - b)

                wait_idx(b)

                @pl.when(i >= 2)
                def _():
                    wait_wb(b)

                run_gather(b)
                issue_wb(i, b)
            return carry

        lax.fori_loop(0, per_w // 2, body, 0)
        if per_w % 2:
            i = per_w - 1
            b = i % 2
            wait_idx(b)
            wait_wb(b)
            run_gather(b)
            issue_wb(i, b)
        wait_wb(0)
        wait_wb(1)
        if rem:
            @pl.when(wid < rem)
            def _():
                off = pl.multiple_of((chunk0 + per_w * NW) * CHUNK
                                     + wid * CHUNK, CHUNK)
                pltpu.sync_copy(src_hbm.at[pl.ds(off, CHUNK)], isrc[0])
                pltpu.sync_copy(dst_hbm.at[pl.ds(off, CHUNK)], idst[0])
                run_gather(0)
                for t in range(nt):
                    pltpu.sync_copy(bufs[t][0], outs[t].at[pl.ds(off, CHUNK)])

    return k(src, dst, *tables)


def _sc_scatter(v, idx, zeros_big, chunk0=0, n_chunks=None):
    e, w = v.shape
    n = zeros_big.shape[0]
    if n_chunks is None:
        n_chunks = e // CHUNK
    per_w = n_chunks // NW
    rem = n_chunks - per_w * NW
    rows = (n // NS) // 8 * 8
    tail = n - NS * rows
    out_type = tuple(jax.ShapeDtypeStruct((n, w), jnp.float32)
                     for _ in range(NC))
    assert per_w >= 3
    scratch = [pltpu.VMEM((CHUNK,), jnp.int32)] * 2 + \
              [pltpu.VMEM((CHUNK, w), jnp.float32)] * 2 + \
              [pltpu.VMEM_SHARED((n, w), jnp.float32)] + \
              [pltpu.SemaphoreType.DMA] * 4

    @functools.partial(pl.kernel, out_type=out_type, mesh=_mesh(),
                       scratch_types=scratch)
    def k(v_hbm, idx_hbm, z_hbm, out0, out1, ibuf0, ibuf1, vbuf0, vbuf1, acc,
          seml0, seml1, sema0, sema1):
        ibuf = (ibuf0, ibuf1)
        vbuf = (vbuf0, vbuf1)
        seml = (seml0, seml1)
        sema = (sema0, sema1)
        cid = lax.axis_index("c")
        sid = lax.axis_index("s")
        wid = sid * NC + cid

        @pl.when(sid == 0)
        def _():
            pltpu.sync_copy(z_hbm, acc)

        plsc.subcore_barrier()

        def off_of(i):
            return pl.multiple_of((chunk0 + i * NW) * CHUNK + wid * CHUNK,
                                  CHUNK)

        def issue_load(i, b):
            off = off_of(i)
            pltpu.async_copy(idx_hbm.at[pl.ds(off, CHUNK)], ibuf[b], seml[b])
            pltpu.async_copy(v_hbm.at[pl.ds(off, CHUNK)], vbuf[b], seml[b])

        def wait_load(b):
            pltpu.make_async_copy(idx_hbm.at[pl.ds(0, CHUNK)], ibuf[b],
                                  seml[b]).wait()
            pltpu.make_async_copy(v_hbm.at[pl.ds(0, CHUNK)], vbuf[b],
                                  seml[b]).wait()

        issue_load(0, 0)

        def body(j2, carry):
            for b in range(2):
                i = 2 * j2 + b

                @pl.when(i + 1 < per_w)
                def _():
                    issue_load(i + 1, 1 - b)

                wait_load(b)
                pltpu.sync_copy(vbuf[b], acc.at[ibuf[b]], add=True)
            return carry

        lax.fori_loop(0, per_w // 2, body, 0)
        if per_w % 2:
            b = (per_w - 1) % 2
            wait_load(b)
            pltpu.sync_copy(vbuf[b], acc.at[ibuf[b]], add=True)
        if rem:
            @pl.when(wid < rem)
            def _():
                off = pl.multiple_of((chunk0 + per_w * NW) * CHUNK
                                     + wid * CHUNK, CHUNK)
                pltpu.sync_copy(idx_hbm.at[pl.ds(off, CHUNK)], ibuf[0])
                pltpu.sync_copy(v_hbm.at[pl.ds(off, CHUNK)], vbuf[0])
                pltpu.sync_copy(vbuf[0], acc.at[ibuf[0]], add=True)
        plsc.subcore_barrier()

        r0 = pl.multiple_of(sid * rows, 8)

        @pl.when(cid == 0)
        def _():
            pltpu.sync_copy(acc.at[pl.ds(r0, rows)], out0.at[pl.ds(r0, rows)])

            @pl.when(sid == 0)
            def _():
                if tail:
                    pltpu.sync_copy(acc.at[pl.ds(NS * rows, tail)],
                                    out0.at[pl.ds(NS * rows, tail)])

        @pl.when(cid == 1)
        def _():
            pltpu.sync_copy(acc.at[pl.ds(r0, rows)], out1.at[pl.ds(r0, rows)])

            @pl.when(sid == 0)
            def _():
                if tail:
                    pltpu.sync_copy(acc.at[pl.ds(NS * rows, tail)],
                                    out1.at[pl.ds(NS * rows, tail)])

    return k(v, idx, zeros_big)


def _sc_counts(idx, zeros_big, ones_chunk):
    e = idx.shape[0]
    n, w = zeros_big.shape
    n_chunks = e // CHUNK
    per_w = n_chunks // NW
    rem = n_chunks - per_w * NW
    rows = (n // NS) // 8 * 8
    tail = n - NS * rows
    out_type = tuple(jax.ShapeDtypeStruct((n, w), jnp.float32)
                     for _ in range(NC))
    assert per_w % 2 == 0
    scratch = [pltpu.VMEM((CHUNK,), jnp.int32)] * 2 + \
              [pltpu.VMEM((CHUNK, w), jnp.float32)] + \
              [pltpu.VMEM_SHARED((n, w), jnp.float32)] + \
              [pltpu.SemaphoreType.DMA] * 4

    @functools.partial(pl.kernel, out_type=out_type, mesh=_mesh(),
                       scratch_types=scratch)
    def k(idx_hbm, z_hbm, ones_hbm, out0, out1, ibuf0, ibuf1, obuf, acc,
          seml0, seml1, sema0, sema1):
        ibuf = (ibuf0, ibuf1)
        seml = (seml0, seml1)
        sema = (sema0, sema1)
        cid = lax.axis_index("c")
        sid = lax.axis_index("s")
        wid = sid * NC + cid

        @pl.when(sid == 0)
        def _():
            pltpu.sync_copy(z_hbm, acc)

        pltpu.sync_copy(ones_hbm, obuf)
        plsc.subcore_barrier()

        def off_of(i):
            return pl.multiple_of((i * NW + wid) * CHUNK, CHUNK)

        def issue_load(i, b):
            pltpu.async_copy(idx_hbm.at[pl.ds(off_of(i), CHUNK)], ibuf[b],
                             seml[b])

        def wait_load(b):
            pltpu.make_async_copy(idx_hbm.at[pl.ds(0, CHUNK)], ibuf[b],
                                  seml[b]).wait()

        issue_load(0, 0)

        def body(j2, carry):
            for b in range(2):
                i = 2 * j2 + b

                @pl.when(i + 1 < per_w)
                def _():
                    issue_load(i + 1, 1 - b)

                wait_load(b)
                pltpu.sync_copy(obuf, acc.at[ibuf[b]], add=True)
            return carry

        lax.fori_loop(0, per_w // 2, body, 0)
        if rem:
            @pl.when(wid < rem)
            def _():
                off = pl.multiple_of((per_w * NW + wid) * CHUNK, CHUNK)
                pltpu.sync_copy(idx_hbm.at[pl.ds(off, CHUNK)], ibuf[0])
                pltpu.sync_copy(obuf, acc.at[ibuf[0]], add=True)
        plsc.subcore_barrier()

        r0 = pl.multiple_of(sid * rows, 8)

        @pl.when(cid == 0)
        def _():
            pltpu.sync_copy(acc.at[pl.ds(r0, rows)], out0.at[pl.ds(r0, rows)])

            @pl.when(sid == 0)
            def _():
                if tail:
                    pltpu.sync_copy(acc.at[pl.ds(NS * rows, tail)],
                                    out0.at[pl.ds(NS * rows, tail)])

        @pl.when(cid == 1)
        def _():
            pltpu.sync_copy(acc.at[pl.ds(r0, rows)], out1.at[pl.ds(r0, rows)])

            @pl.when(sid == 0)
            def _():
                if tail:
                    pltpu.sync_copy(acc.at[pl.ds(NS * rows, tail)],
                                    out1.at[pl.ds(NS * rows, tail)])

    return k(idx, zeros_big, ones_chunk)


def kernel(nodes, x, edges, virtual, edge_index, node_batch, lengths, t, cond,
           params):
    n = nodes.shape[0]
    src = edge_index[0]
    dst = edge_index[1]

    (w1p, b1p), (w2p, b2p) = params["pe"]
    wn_p = w1p[:128]
    wx_p = jnp.pad(w1p[128:], ((0, 128 - (w1p.shape[0] - 128)), (0, 0)))
    xp = jnp.pad(x, ((0, 0), (0, 128 - x.shape[1])))
    nb2 = node_batch.reshape(n, 1)

    def lin(layer):
        w, b = layer
        return w, b.reshape(1, -1)

    e0w1, e0b1 = lin(params["mpnn0"]["edge"][0])
    e0w2, e0b2 = lin(params["mpnn0"]["edge"][1])
    n0w1, n0b1 = lin(params["mpnn0"]["node"][0])
    n0w2, n0b2 = lin(params["mpnn0"]["node"][1])
    g0w1, g0b1 = lin(params["mpnn0"]["glob"][0])
    g0w2, g0b2 = lin(params["mpnn0"]["glob"][1])
    e1w1, e1b1 = lin(params["mpnn1"]["edge"][0])
    e1w2, e1b2 = lin(params["mpnn1"]["edge"][1])
    n1w1, n1b1 = lin(params["mpnn1"]["node"][0])
    n1w2, n1b2 = lin(params["mpnn1"]["node"][1])
    g1w1, g1b1 = lin(params["mpnn1"]["glob"][0])
    g1w2, g1b2 = lin(params["mpnn1"]["glob"][1])
    ow1, ob1 = lin(params["out"][0])
    ow2, ob2 = lin(params["out"][1])

    zeros_big = jnp.zeros((n, 128), jnp.float32)
    ones_chunk = jnp.ones((CHUNK, 128), jnp.float32)

    h0, ts0, td0, ohn = _run_pre(
        nodes, xp, nb2, wn_p, wx_p, b1p.reshape(1, -1), w2p,
        b2p.reshape(1, -1), e0w1[0:128], e0w1[128:256], e0b1)

    c0, c1 = _sc_counts(dst, zeros_big, ones_chunk)
    si0, si1 = _sc_scatter(edges, dst, zeros_big)

    gs0, gd0 = _sc_gather(src, dst, [ts0, td0], [0, 1])
    edges1, sum_eb0, cnt_eb = _run_edge(
        False, gs0, gd0, edges, e0w1[256:384], e0w2, e0b2)
    t10, t11 = _sc_scatter(edges1, dst, zeros_big)
    h1, sum_n0, cnt_n, ts1, td1 = _run_node(
        True, h0, [t10, t11], [si0, si1], c0, c1, ohn,
        n0w1[0:128], n0w1[128:256], n0b1, n0w2, n0b2,
        [e1w1[0:128], e1w1[128:256], e1b1])
    virtual1 = _run_glob(sum_n0, cnt_n, sum_eb0, cnt_eb, virtual,
                         g0w1[0:128], g0w1[128:256], g0w1[256:384], g0b1,
                         g0w2, g0b2)

    gs1, gd1 = _sc_gather(src, dst, [ts1, td1], [0, 1])
    edges2, sum_eb1 = _run_edge(
        True, gs1, gd1, edges1, e1w1[256:384], e1w2, e1b2,
        wv=e1w1[384:512], virt=virtual1)
    t20, t21 = _sc_scatter(edges2, dst, zeros_big)
    h2, sum_n1 = _run_node(
        False, h1, [t20, t21], [t10, t11],
        c0, c1, ohn, n1w1[0:128], n1w1[128:256], n1b1, n1w2, n1b2,
        [n1w1[256:384], virtual1])
    virtual2 = _run_glob(sum_n1, cnt_n, sum_eb1, cnt_eb, virtual1,
                         g1w1[0:128], g1w1[128:256], g1w1[256:384], g1b1,
                         g1w2, g1b2)

    ret = _run_out(h2, [t20, t21], c0, c1, ohn, virtual2,
                   t, cond, ow1[0:128], ow1[128:256], ow1[256:384],
                   ow1[384:512], ow1[512:576], ob1, ow2, ob2)

    return ((h2, edges2, virtual2, edge_index, node_batch, lengths, t, cond),
            ret)
